# baseline (speedup 1.0000x reference)
import functools

import jax
import jax.numpy as jnp
from jax import lax
from jax.experimental import pallas as pl
from jax.experimental.pallas import tpu as pltpu
from jax.experimental.pallas import tpu_sc as plsc

N = 10000
B = 8
E = 128
H = 8
D_H = 16
L = 16
NE = 320000

NW = 32
EPT = NE // NW
CH = 80
NCH = EPT // CH
NPAD = 10240
RB = 1000
NBLK = N // RB

_sc_mesh = dict(core_axis_name="c", subcore_axis_name="s")


def _deg_body(dst_hbm, out_hbm, idx_v, acc_v):
    cid = lax.axis_index("c")
    sid = lax.axis_index("s")
    wid = sid * 2 + cid
    pltpu.sync_copy(dst_hbm.at[pl.ds(wid * EPT, EPT)], idx_v)
    zeros16 = jnp.zeros((16,), jnp.float32)
    ones16 = jnp.ones((16,), jnp.float32)

    def zero(k, c):
        acc_v[pl.ds(k * 16, 16)] = zeros16
        return c

    lax.fori_loop(0, EPT // 16, zero, 0)

    def add(k, c):
        dvec = idx_v[pl.ds(k * 16, 16)]
        plsc.addupdate_scatter(acc_v, [dvec], ones16)
        return c

    lax.fori_loop(0, EPT // 16, add, 0)
    pltpu.sync_copy(acc_v, out_hbm.at[wid])


@functools.cache
def _deg():
    return functools.partial(
        pl.kernel,
        mesh=plsc.VectorSubcoreMesh(**_sc_mesh),
        out_type=jax.ShapeDtypeStruct((NW, N), jnp.float32),
        scratch_types=[
            pltpu.VMEM((EPT,), jnp.int32),
            pltpu.VMEM((N,), jnp.float32),
        ],
        compiler_params=pltpu.CompilerParams(needs_layout_passes=False),
    )(_deg_body)


def _scale_body(dp_ref, x_ref, y_ref, dinvb_ref):
    ones = jnp.ones((NW, 1), jnp.float32)
    deg = lax.dot_general(dp_ref[...], ones, (((0,), (0,)), ((), ())),
                          preferred_element_type=jnp.float32) + 1.0
    dinv = lax.rsqrt(deg)
    y_ref[...] = dinv * x_ref[...]
    dinvb_ref[...] = jnp.broadcast_to(dinv, (N, E))


def _scale(dp, x):
    return pl.pallas_call(
        _scale_body,
        out_shape=(
            jax.ShapeDtypeStruct((N, E), jnp.float32),
            jax.ShapeDtypeStruct((N, E), jnp.float32),
        ),
    )(dp, x)


def _agg_body(src_hbm, dst_hbm, y_hbm, out_hbm,
              sall_v, didx0_v, didx1_v, rows0_v, rows1_v, shacc,
              gsem0, gsem1, isem0, isem1, ssem0, ssem1):
    cid = lax.axis_index("c")
    sid = lax.axis_index("s")
    wid = sid * 2 + cid
    base = wid * EPT
    zeros16 = jnp.zeros((16,), jnp.float32)

    def zero(k, c):
        rows0_v[k // 8, pl.ds((k % 8) * 16, 16)] = zeros16
        return c

    lax.fori_loop(0, CH * 8, zero, 0)
    zbase = pl.multiple_of(sid * 640, 8)
    for b in range(8):
        pltpu.sync_copy(rows0_v, shacc.at[pl.ds(zbase + b * CH, CH)])
    plsc.subcore_barrier()

    pltpu.sync_copy(src_hbm.at[pl.ds(base, EPT)], sall_v)

    def issue(ci, didx_v, rows_v, gsem, isem):
        off = pl.multiple_of(ci * CH, 8)
        pltpu.async_copy(dst_hbm.at[pl.ds(base + off, CH)], didx_v, isem)
        pltpu.async_copy(y_hbm.at[sall_v.at[pl.ds(off, CH)]], rows_v, gsem)

    def scat(ci, didx_v, rows_v, gsem, isem, ssem):
        off = pl.multiple_of(ci * CH, 8)
        pltpu.make_async_copy(dst_hbm.at[pl.ds(base + off, CH)],
                              didx_v, isem).wait()
        pltpu.make_async_copy(y_hbm.at[sall_v.at[pl.ds(off, CH)]],
                              rows_v, gsem).wait()
        pltpu.async_copy(rows_v, shacc.at[didx_v], ssem, add=True)

    def swait(rows_v, didx_v, ssem):
        pltpu.make_async_copy(rows_v, shacc.at[didx_v], ssem).wait()

    issue(0, didx0_v, rows0_v, gsem0, isem0)

    def chunk(ci, c):
        @pl.when(ci % 2 == 0)
        def _even():
            scat(ci, didx0_v, rows0_v, gsem0, isem0, ssem0)

            @pl.when(ci + 1 < NCH)
            def _pref():
                @pl.when(ci >= 1)
                def _w():
                    swait(rows1_v, didx1_v, ssem1)
                issue(ci + 1, didx1_v, rows1_v, gsem1, isem1)

        @pl.when(ci % 2 == 1)
        def _odd():
            scat(ci, didx1_v, rows1_v, gsem1, isem1, ssem1)

            @pl.when(ci + 1 < NCH)
            def _pref():
                swait(rows0_v, didx0_v, ssem0)
                issue(ci + 1, didx0_v, rows0_v, gsem0, isem0)

        return c

    lax.fori_loop(0, NCH, chunk, 0)
    swait(rows0_v, didx0_v, ssem0)
    swait(rows1_v, didx1_v, ssem1)
    plsc.subcore_barrier()

    obase = pl.multiple_of(sid * 624, 8)

    @pl.when(sid < 15)
    def _cp():
        pltpu.sync_copy(shacc.at[pl.ds(obase, 624)],
                        out_hbm.at[cid, pl.ds(obase, 624)])

    @pl.when(sid == 15)
    def _cp_tail():
        pltpu.sync_copy(shacc.at[pl.ds(15 * 624, 640)],
                        out_hbm.at[cid, pl.ds(15 * 624, 640)])


@functools.cache
def _agg():
    return functools.partial(
        pl.kernel,
        mesh=plsc.VectorSubcoreMesh(**_sc_mesh),
        out_type=jax.ShapeDtypeStruct((2, N, E), jnp.float32),
        scratch_types=[
            pltpu.VMEM((EPT,), jnp.int32),
            pltpu.VMEM((CH,), jnp.int32),
            pltpu.VMEM((CH,), jnp.int32),
            pltpu.VMEM((CH, E), jnp.float32),
            pltpu.VMEM((CH, E), jnp.float32),
            pltpu.VMEM_SHARED((NPAD, E), jnp.float32),
            pltpu.SemaphoreType.DMA,
            pltpu.SemaphoreType.DMA,
            pltpu.SemaphoreType.DMA,
            pltpu.SemaphoreType.DMA,
            pltpu.SemaphoreType.DMA,
            pltpu.SemaphoreType.DMA,
        ],
    )(_agg_body)


def _dense_body(aggp_ref, y_ref, dinvb_ref, batch_ref, seed_ref, qp_ref,
                wk_ref, bk_ref, wv_ref, bv_ref, ow_ref, ob_ref,
                fw_ref, fb_ref, l1g_ref, l1b_ref, l2g_ref, l2b_ref,
                out_ref, qt_ref, m_ref, den_ref, num_ref, cnt_ref):
    i = pl.program_id(0)
    f32 = jnp.float32

    @pl.when(i == 0)
    def _init():
        m_ref[...] = jnp.full((B, E), -1e30, f32)
        den_ref[...] = jnp.zeros((B, E), f32)
        num_ref[...] = jnp.zeros((B, E, E), f32)
        cnt_ref[...] = jnp.zeros((B, E), f32)
        q2 = lax.dot_general(seed_ref[...], qp_ref[...],
                             (((1,), (1,)), ((), ())),
                             preferred_element_type=f32) * (D_H ** -0.5)
        rep = jnp.concatenate([q2] * H, axis=0)
        rh = lax.broadcasted_iota(jnp.int32, (E, E), 0) // D_H
        eh = lax.broadcasted_iota(jnp.int32, (E, E), 1) // D_H
        qt_ref[...] = jnp.where(rh == eh, rep, 0.0)

    pre = dinvb_ref[...] * (aggp_ref[0] + aggp_ref[1] + y_ref[...])
    k = lax.dot_general(pre, wk_ref[...], (((1,), (1,)), ((), ())),
                        preferred_element_type=f32) + bk_ref[...]
    v = lax.dot_general(pre, wv_ref[...], (((1,), (1,)), ((), ())),
                        preferred_element_type=f32) + bv_ref[...]
    logits = lax.dot_general(k, qt_ref[...], (((1,), (1,)), ((), ())),
                             preferred_element_type=f32)

    b_blk = batch_ref[0, 0, :]
    gid = lax.broadcasted_iota(jnp.int32, (RB, B), 1)
    bm = b_blk[:, None] == gid
    bmf = bm.astype(f32)
    cnt_ref[...] = cnt_ref[...] + jnp.sum(bmf, axis=0)[:, None]

    for g in range(B):
        mk = bm[:, g:g + 1]

        @pl.when(jnp.sum(bmf[:, g]) > 0.0)
        def _upd(g=g, mk=mk):
            lg = jnp.where(mk, logits, -1e30)
            bmax = jnp.max(lg, axis=0)
            mold = m_ref[g, :]
            mnew = jnp.maximum(mold, bmax)
            adj = jnp.exp(mold - mnew)
            w = jnp.where(mk, jnp.exp(logits - mnew[None, :]), 0.0)
            den_ref[g, :] = den_ref[g, :] * adj + jnp.sum(w, axis=0)
            num_ref[g, :, :] = (num_ref[g, :, :] * adj[:, None] +
                                lax.dot_general(w, v,
                                                (((0,), (0,)), ((), ())),
                                                preferred_element_type=f32))
            m_ref[g, :] = mnew

    @pl.when(i == NBLK - 1)
    def _fin():
        counts = cnt_ref[...][:, 0]
        nmax = jnp.max(counts)
        pad = nmax - counts
        mfin = m_ref[...]
        padterm = jnp.where(pad[:, None] > 0.0,
                            pad[:, None] * jnp.exp(-mfin), 0.0)
        denf = den_ref[...] + padterm

        rows = []
        for g in range(B):
            segs = []
            for h in range(H):
                sub = num_ref[g, h * D_H:(h + 1) * D_H,
                              h * D_H:(h + 1) * D_H]
                dsub = denf[g, h * D_H:(h + 1) * D_H]
                segs.append(sub / dsub[:, None])
            rows.append(jnp.concatenate(segs, axis=1))
        ao = jnp.concatenate(rows, axis=0)

        qm = jnp.broadcast_to(seed_ref[...][None, :, :],
                              (B, L, E)).reshape(B * L, E)
        ao2 = lax.dot_general(ao, ow_ref[...], (((1,), (1,)), ((), ())),
                              preferred_element_type=f32) + ob_ref[...]
        s1 = qm + ao2
        mu1 = jnp.mean(s1, axis=1, keepdims=True)
        va1 = jnp.mean((s1 - mu1) ** 2, axis=1, keepdims=True)
        e1 = ((s1 - mu1) / jnp.sqrt(va1 + 1e-5)) * l1g_ref[...] + l1b_ref[...]
        ff = lax.dot_general(e1, fw_ref[...], (((1,), (1,)), ((), ())),
                             preferred_element_type=f32) + fb_ref[...]
        s2 = e1 + ff
        mu2 = jnp.mean(s2, axis=1, keepdims=True)
        va2 = jnp.mean((s2 - mu2) ** 2, axis=1, keepdims=True)
        e2 = ((s2 - mu2) / jnp.sqrt(va2 + 1e-5)) * l2g_ref[...] + l2b_ref[...]
        out_ref[...] = e2.reshape(B, L, E)


def _full(shape):
    nd = len(shape)
    return pl.BlockSpec(shape, lambda i, _n=nd: (0,) * _n)


def _dense(aggp, y, dinvb, batch3, seedf, qp, wkf, bkf, wvf, bvf,
           ow, ob, fw, fb, l1g, l1b, l2g, l2b):
    return pl.pallas_call(
        _dense_body,
        grid=(NBLK,),
        in_specs=[
            pl.BlockSpec((2, RB, E), lambda i: (0, i, 0)),
            pl.BlockSpec((RB, E), lambda i: (i, 0)),
            pl.BlockSpec((RB, E), lambda i: (i, 0)),
            pl.BlockSpec((1, 1, RB), lambda i: (i, 0, 0)),
            _full((L, E)), _full((E, E)),
            _full((E, E)), _full((1, E)), _full((E, E)), _full((1, E)),
            _full((E, E)), _full((1, E)), _full((E, E)), _full((1, E)),
            _full((1, E)), _full((1, E)), _full((1, E)), _full((1, E)),
        ],
        out_specs=pl.BlockSpec((B, L, E), lambda i: (0, 0, 0)),
        out_shape=jax.ShapeDtypeStruct((B, L, E), jnp.float32),
        scratch_shapes=[
            pltpu.VMEM((E, E), jnp.float32),
            pltpu.VMEM((B, E), jnp.float32),
            pltpu.VMEM((B, E), jnp.float32),
            pltpu.VMEM((B, E, E), jnp.float32),
            pltpu.VMEM((B, E), jnp.float32),
        ],
        compiler_params=pltpu.CompilerParams(
            dimension_semantics=("arbitrary",)),
    )(aggp, y, dinvb, batch3, seedf, qp, wkf, bkf, wvf, bvf,
      ow, ob, fw, fb, l1g, l1b, l2g, l2b)


def kernel(x_dense, x, edge_index, batch, seed, Wk, bk, Wv, bv,
           q_proj_weight, out_w, out_b, ff_w, ff_b,
           ln1_g, ln1_b, ln2_g, ln2_b):
    src = edge_index[0]
    dst = edge_index[1]

    deg_part = _deg()(dst)
    y, dinvb = _scale(deg_part, x)
    aggp = _agg()(src, dst, y)

    return _dense(
        aggp, y, dinvb,
        batch.reshape(NBLK, 1, RB),
        seed.reshape(L, E), q_proj_weight,
        Wk.reshape(E, E), bk.reshape(1, E),
        Wv.reshape(E, E), bv.reshape(1, E),
        out_w, out_b.reshape(1, E), ff_w, ff_b.reshape(1, E),
        ln1_g.reshape(1, E), ln1_b.reshape(1, E),
        ln2_g.reshape(1, E), ln2_b.reshape(1, E),
    )


# speedup vs baseline: 1.1883x; 1.1883x over previous
import functools

import jax
import jax.numpy as jnp
from jax import lax
from jax.experimental import pallas as pl
from jax.experimental.pallas import tpu as pltpu
from jax.experimental.pallas import tpu_sc as plsc

N = 10000
B = 8
E = 128
H = 8
D_H = 16
L = 16
NE = 320000

NW = 32
EPT = NE // NW
CH = 80
NCH = EPT // CH
NPAD = 10240
RB = 1000
NBLK = N // RB

_sc_mesh = dict(core_axis_name="c", subcore_axis_name="s")


def _deg_body(dst_hbm, out_hbm, idx_v, acc_v):
    cid = lax.axis_index("c")
    sid = lax.axis_index("s")
    wid = sid * 2 + cid
    pltpu.sync_copy(dst_hbm.at[pl.ds(wid * EPT, EPT)], idx_v)
    zeros16 = jnp.zeros((16,), jnp.float32)
    ones16 = jnp.ones((16,), jnp.float32)

    def zero(k, c):
        acc_v[pl.ds(k * 16, 16)] = zeros16
        return c

    lax.fori_loop(0, EPT // 16, zero, 0)

    def add(k, c):
        dvec = idx_v[pl.ds(k * 16, 16)]
        plsc.addupdate_scatter(acc_v, [dvec], ones16)
        return c

    lax.fori_loop(0, EPT // 16, add, 0)
    pltpu.sync_copy(acc_v, out_hbm.at[wid])


@functools.cache
def _deg():
    return functools.partial(
        pl.kernel,
        mesh=plsc.VectorSubcoreMesh(**_sc_mesh),
        out_type=jax.ShapeDtypeStruct((NW, N), jnp.float32),
        scratch_types=[
            pltpu.VMEM((EPT,), jnp.int32),
            pltpu.VMEM((N,), jnp.float32),
        ],
        compiler_params=pltpu.CompilerParams(needs_layout_passes=False),
    )(_deg_body)


def _scale_body(dp_ref, x_ref, y_ref, dinvb_ref):
    ones = jnp.ones((NW, 1), jnp.float32)
    deg = lax.dot_general(dp_ref[...], ones, (((0,), (0,)), ((), ())),
                          preferred_element_type=jnp.float32) + 1.0
    dinv = lax.rsqrt(deg)
    y_ref[...] = dinv * x_ref[...]
    dinvb_ref[...] = jnp.broadcast_to(dinv, (N, E))


def _scale(dp, x):
    return pl.pallas_call(
        _scale_body,
        out_shape=(
            jax.ShapeDtypeStruct((N, E), jnp.float32),
            jax.ShapeDtypeStruct((N, E), jnp.float32),
        ),
    )(dp, x)


def _agg_body(src_hbm, dst_hbm, y_hbm, out_hbm,
              sall_v, didx0_v, didx1_v, rows0_v, rows1_v, shacc,
              gsem0, gsem1, isem0, isem1):
    cid = lax.axis_index("c")
    sid = lax.axis_index("s")
    wid = sid * 2 + cid
    base = wid * EPT
    zeros16 = jnp.zeros((16,), jnp.float32)

    def zero(k, c):
        rows0_v[k // 8, pl.ds((k % 8) * 16, 16)] = zeros16
        return c

    lax.fori_loop(0, CH * 8, zero, 0)
    zbase = pl.multiple_of(sid * 640, 8)
    for b in range(8):
        pltpu.sync_copy(rows0_v, shacc.at[pl.ds(zbase + b * CH, CH)])
    plsc.subcore_barrier()

    pltpu.sync_copy(src_hbm.at[pl.ds(base, EPT)], sall_v)

    def issue(ci, didx_v, rows_v, gsem, isem):
        off = pl.multiple_of(ci * CH, 8)
        pltpu.async_copy(dst_hbm.at[pl.ds(base + off, CH)], didx_v, isem)
        pltpu.async_copy(y_hbm.at[sall_v.at[pl.ds(off, CH)]], rows_v, gsem)

    def drain(ci, didx_v, rows_v, gsem, isem):
        off = pl.multiple_of(ci * CH, 8)
        pltpu.make_async_copy(dst_hbm.at[pl.ds(base + off, CH)],
                              didx_v, isem).wait()
        pltpu.make_async_copy(y_hbm.at[sall_v.at[pl.ds(off, CH)]],
                              rows_v, gsem).wait()
        pltpu.sync_copy(rows_v, shacc.at[didx_v], add=True)

    issue(0, didx0_v, rows0_v, gsem0, isem0)

    def chunk(ci, c):
        @pl.when(ci % 2 == 0)
        def _even():
            @pl.when(ci + 1 < NCH)
            def _pref():
                issue(ci + 1, didx1_v, rows1_v, gsem1, isem1)
            drain(ci, didx0_v, rows0_v, gsem0, isem0)

        @pl.when(ci % 2 == 1)
        def _odd():
            @pl.when(ci + 1 < NCH)
            def _pref():
                issue(ci + 1, didx0_v, rows0_v, gsem0, isem0)
            drain(ci, didx1_v, rows1_v, gsem1, isem1)

        return c

    lax.fori_loop(0, NCH, chunk, 0)
    plsc.subcore_barrier()

    obase = pl.multiple_of(sid * 624, 8)

    @pl.when(sid < 15)
    def _cp():
        pltpu.sync_copy(shacc.at[pl.ds(obase, 624)],
                        out_hbm.at[cid, pl.ds(obase, 624)])

    @pl.when(sid == 15)
    def _cp_tail():
        pltpu.sync_copy(shacc.at[pl.ds(15 * 624, 640)],
                        out_hbm.at[cid, pl.ds(15 * 624, 640)])


@functools.cache
def _agg():
    return functools.partial(
        pl.kernel,
        mesh=plsc.VectorSubcoreMesh(**_sc_mesh),
        out_type=jax.ShapeDtypeStruct((2, N, E), jnp.float32),
        scratch_types=[
            pltpu.VMEM((EPT,), jnp.int32),
            pltpu.VMEM((CH,), jnp.int32),
            pltpu.VMEM((CH,), jnp.int32),
            pltpu.VMEM((CH, E), jnp.float32),
            pltpu.VMEM((CH, E), jnp.float32),
            pltpu.VMEM_SHARED((NPAD, E), jnp.float32),
            pltpu.SemaphoreType.DMA,
            pltpu.SemaphoreType.DMA,
            pltpu.SemaphoreType.DMA,
            pltpu.SemaphoreType.DMA,
        ],
    )(_agg_body)


def _dense_body(aggp_ref, y_ref, dinvb_ref, batch_ref, seed_ref, qp_ref,
                wk_ref, bk_ref, wv_ref, bv_ref, ow_ref, ob_ref,
                fw_ref, fb_ref, l1g_ref, l1b_ref, l2g_ref, l2b_ref,
                out_ref, qt_ref, m_ref, den_ref, num_ref, cnt_ref):
    i = pl.program_id(0)
    f32 = jnp.float32

    @pl.when(i == 0)
    def _init():
        m_ref[...] = jnp.full((B, E), -1e30, f32)
        den_ref[...] = jnp.zeros((B, E), f32)
        num_ref[...] = jnp.zeros((B, E, E), f32)
        cnt_ref[...] = jnp.zeros((B, E), f32)
        q2 = lax.dot_general(seed_ref[...], qp_ref[...],
                             (((1,), (1,)), ((), ())),
                             preferred_element_type=f32) * (D_H ** -0.5)
        rep = jnp.concatenate([q2] * H, axis=0)
        rh = lax.broadcasted_iota(jnp.int32, (E, E), 0) // D_H
        eh = lax.broadcasted_iota(jnp.int32, (E, E), 1) // D_H
        qt_ref[...] = jnp.where(rh == eh, rep, 0.0)

    pre = dinvb_ref[...] * (aggp_ref[0] + aggp_ref[1] + y_ref[...])
    k = lax.dot_general(pre, wk_ref[...], (((1,), (1,)), ((), ())),
                        preferred_element_type=f32) + bk_ref[...]
    v = lax.dot_general(pre, wv_ref[...], (((1,), (1,)), ((), ())),
                        preferred_element_type=f32) + bv_ref[...]
    logits = lax.dot_general(k, qt_ref[...], (((1,), (1,)), ((), ())),
                             preferred_element_type=f32)

    b_blk = batch_ref[0, 0, :]
    gid = lax.broadcasted_iota(jnp.int32, (RB, B), 1)
    bm = b_blk[:, None] == gid
    bmf = bm.astype(f32)
    cnt_ref[...] = cnt_ref[...] + jnp.sum(bmf, axis=0)[:, None]

    for g in range(B):
        mk = bm[:, g:g + 1]

        @pl.when(jnp.sum(bmf[:, g]) > 0.0)
        def _upd(g=g, mk=mk):
            lg = jnp.where(mk, logits, -1e30)
            bmax = jnp.max(lg, axis=0)
            mold = m_ref[g, :]
            mnew = jnp.maximum(mold, bmax)
            adj = jnp.exp(mold - mnew)
            w = jnp.where(mk, jnp.exp(logits - mnew[None, :]), 0.0)
            den_ref[g, :] = den_ref[g, :] * adj + jnp.sum(w, axis=0)
            num_ref[g, :, :] = (num_ref[g, :, :] * adj[:, None] +
                                lax.dot_general(w, v,
                                                (((0,), (0,)), ((), ())),
                                                preferred_element_type=f32))
            m_ref[g, :] = mnew

    @pl.when(i == NBLK - 1)
    def _fin():
        counts = cnt_ref[...][:, 0]
        nmax = jnp.max(counts)
        pad = nmax - counts
        mfin = m_ref[...]
        padterm = jnp.where(pad[:, None] > 0.0,
                            pad[:, None] * jnp.exp(-mfin), 0.0)
        denf = den_ref[...] + padterm

        rows = []
        for g in range(B):
            segs = []
            for h in range(H):
                sub = num_ref[g, h * D_H:(h + 1) * D_H,
                              h * D_H:(h + 1) * D_H]
                dsub = denf[g, h * D_H:(h + 1) * D_H]
                segs.append(sub / dsub[:, None])
            rows.append(jnp.concatenate(segs, axis=1))
        ao = jnp.concatenate(rows, axis=0)

        qm = jnp.broadcast_to(seed_ref[...][None, :, :],
                              (B, L, E)).reshape(B * L, E)
        ao2 = lax.dot_general(ao, ow_ref[...], (((1,), (1,)), ((), ())),
                              preferred_element_type=f32) + ob_ref[...]
        s1 = qm + ao2
        mu1 = jnp.mean(s1, axis=1, keepdims=True)
        va1 = jnp.mean((s1 - mu1) ** 2, axis=1, keepdims=True)
        e1 = ((s1 - mu1) / jnp.sqrt(va1 + 1e-5)) * l1g_ref[...] + l1b_ref[...]
        ff = lax.dot_general(e1, fw_ref[...], (((1,), (1,)), ((), ())),
                             preferred_element_type=f32) + fb_ref[...]
        s2 = e1 + ff
        mu2 = jnp.mean(s2, axis=1, keepdims=True)
        va2 = jnp.mean((s2 - mu2) ** 2, axis=1, keepdims=True)
        e2 = ((s2 - mu2) / jnp.sqrt(va2 + 1e-5)) * l2g_ref[...] + l2b_ref[...]
        out_ref[...] = e2.reshape(B, L, E)


def _full(shape):
    nd = len(shape)
    return pl.BlockSpec(shape, lambda i, _n=nd: (0,) * _n)


def _dense(aggp, y, dinvb, batch3, seedf, qp, wkf, bkf, wvf, bvf,
           ow, ob, fw, fb, l1g, l1b, l2g, l2b):
    return pl.pallas_call(
        _dense_body,
        grid=(NBLK,),
        in_specs=[
            pl.BlockSpec((2, RB, E), lambda i: (0, i, 0)),
            pl.BlockSpec((RB, E), lambda i: (i, 0)),
            pl.BlockSpec((RB, E), lambda i: (i, 0)),
            pl.BlockSpec((1, 1, RB), lambda i: (i, 0, 0)),
            _full((L, E)), _full((E, E)),
            _full((E, E)), _full((1, E)), _full((E, E)), _full((1, E)),
            _full((E, E)), _full((1, E)), _full((E, E)), _full((1, E)),
            _full((1, E)), _full((1, E)), _full((1, E)), _full((1, E)),
        ],
        out_specs=pl.BlockSpec((B, L, E), lambda i: (0, 0, 0)),
        out_shape=jax.ShapeDtypeStruct((B, L, E), jnp.float32),
        scratch_shapes=[
            pltpu.VMEM((E, E), jnp.float32),
            pltpu.VMEM((B, E), jnp.float32),
            pltpu.VMEM((B, E), jnp.float32),
            pltpu.VMEM((B, E, E), jnp.float32),
            pltpu.VMEM((B, E), jnp.float32),
        ],
        compiler_params=pltpu.CompilerParams(
            dimension_semantics=("arbitrary",)),
    )(aggp, y, dinvb, batch3, seedf, qp, wkf, bkf, wvf, bvf,
      ow, ob, fw, fb, l1g, l1b, l2g, l2b)


def kernel(x_dense, x, edge_index, batch, seed, Wk, bk, Wv, bv,
           q_proj_weight, out_w, out_b, ff_w, ff_b,
           ln1_g, ln1_b, ln2_g, ln2_b):
    src = edge_index[0]
    dst = edge_index[1]

    deg_part = _deg()(dst)
    y, dinvb = _scale(deg_part, x)
    aggp = _agg()(src, dst, y)

    return _dense(
        aggp, y, dinvb,
        batch.reshape(NBLK, 1, RB),
        seed.reshape(L, E), q_proj_weight,
        Wk.reshape(E, E), bk.reshape(1, E),
        Wv.reshape(E, E), bv.reshape(1, E),
        out_w, out_b.reshape(1, E), ff_w, ff_b.reshape(1, E),
        ln1_g.reshape(1, E), ln1_b.reshape(1, E),
        ln2_g.reshape(1, E), ln2_b.reshape(1, E),
    )


# speedup vs baseline: 1.2454x; 1.0481x over previous
import functools

import jax
import jax.numpy as jnp
from jax import lax
from jax.experimental import pallas as pl
from jax.experimental.pallas import tpu as pltpu
from jax.experimental.pallas import tpu_sc as plsc

N = 10000
B = 8
E = 128
H = 8
D_H = 16
L = 16
NE = 320000

NW = 32
EPT = NE // NW
CH = 128
NCH = EPT // CH
TAIL = EPT - NCH * CH
NPAD = 10240
RB = 1000
NBLK = N // RB

_sc_mesh = dict(core_axis_name="c", subcore_axis_name="s")


def _deg_body(dst_hbm, out_hbm, idx_v, acc_v):
    cid = lax.axis_index("c")
    sid = lax.axis_index("s")
    wid = sid * 2 + cid
    pltpu.sync_copy(dst_hbm.at[pl.ds(wid * EPT, EPT)], idx_v)
    zeros16 = jnp.zeros((16,), jnp.float32)
    ones16 = jnp.ones((16,), jnp.float32)

    def zero(k, c):
        acc_v[pl.ds(k * 16, 16)] = zeros16
        return c

    lax.fori_loop(0, EPT // 16, zero, 0)

    def add(k, c):
        dvec = idx_v[pl.ds(k * 16, 16)]
        plsc.addupdate_scatter(acc_v, [dvec], ones16)
        return c

    lax.fori_loop(0, EPT // 16, add, 0)
    pltpu.sync_copy(acc_v, out_hbm.at[wid])


@functools.cache
def _deg():
    return functools.partial(
        pl.kernel,
        mesh=plsc.VectorSubcoreMesh(**_sc_mesh),
        out_type=jax.ShapeDtypeStruct((NW, N), jnp.float32),
        scratch_types=[
            pltpu.VMEM((EPT,), jnp.int32),
            pltpu.VMEM((N,), jnp.float32),
        ],
        compiler_params=pltpu.CompilerParams(needs_layout_passes=False),
    )(_deg_body)


def _scale_body(dp_ref, x_ref, y_ref, dinvb_ref):
    ones = jnp.ones((NW, 1), jnp.float32)
    deg = lax.dot_general(dp_ref[...], ones, (((0,), (0,)), ((), ())),
                          preferred_element_type=jnp.float32) + 1.0
    dinv = lax.rsqrt(deg)
    y_ref[...] = dinv * x_ref[...]
    dinvb_ref[...] = jnp.broadcast_to(dinv, (N, E))


def _scale(dp, x):
    return pl.pallas_call(
        _scale_body,
        out_shape=(
            jax.ShapeDtypeStruct((N, E), jnp.float32),
            jax.ShapeDtypeStruct((N, E), jnp.float32),
        ),
    )(dp, x)


def _agg_body(src_hbm, dst_hbm, y_hbm, out_hbm,
              sall_v, didx0_v, didx1_v, didxt_v, rows0_v, rows1_v, shacc,
              gsem0, gsem1, isem0, isem1):
    cid = lax.axis_index("c")
    sid = lax.axis_index("s")
    wid = sid * 2 + cid
    base = wid * EPT
    zeros16 = jnp.zeros((16,), jnp.float32)

    def zero(k, c):
        rows0_v[k // 8, pl.ds((k % 8) * 16, 16)] = zeros16
        return c

    lax.fori_loop(0, CH * 8, zero, 0)
    zbase = pl.multiple_of(sid * 640, 8)
    for b in range(5):
        pltpu.sync_copy(rows0_v, shacc.at[pl.ds(zbase + b * CH, CH)])
    plsc.subcore_barrier()

    pltpu.sync_copy(src_hbm.at[pl.ds(base, EPT)], sall_v)

    def issue(ci, didx_v, rows_v, gsem, isem):
        off = pl.multiple_of(ci * CH, 8)
        pltpu.async_copy(dst_hbm.at[pl.ds(base + off, CH)], didx_v, isem)
        pltpu.async_copy(y_hbm.at[sall_v.at[pl.ds(off, CH)]], rows_v, gsem)

    def drain(ci, didx_v, rows_v, gsem, isem):
        off = pl.multiple_of(ci * CH, 8)
        pltpu.make_async_copy(dst_hbm.at[pl.ds(base + off, CH)],
                              didx_v, isem).wait()
        pltpu.make_async_copy(y_hbm.at[sall_v.at[pl.ds(off, CH)]],
                              rows_v, gsem).wait()
        pltpu.sync_copy(rows_v, shacc.at[didx_v], add=True)

    issue(0, didx0_v, rows0_v, gsem0, isem0)

    def chunk(ci, c):
        @pl.when(ci % 2 == 0)
        def _even():
            @pl.when(ci + 1 < NCH)
            def _pref():
                issue(ci + 1, didx1_v, rows1_v, gsem1, isem1)
            drain(ci, didx0_v, rows0_v, gsem0, isem0)

        @pl.when(ci % 2 == 1)
        def _odd():
            @pl.when(ci + 1 < NCH)
            def _pref():
                issue(ci + 1, didx0_v, rows0_v, gsem0, isem0)
            drain(ci, didx1_v, rows1_v, gsem1, isem1)

        return c

    lax.fori_loop(0, NCH, chunk, 0)

    toff = pl.multiple_of(NCH * CH, 8)
    pltpu.sync_copy(dst_hbm.at[pl.ds(base + toff, TAIL)], didxt_v)
    pltpu.async_copy(y_hbm.at[sall_v.at[pl.ds(toff, TAIL)]],
                     rows0_v.at[pl.ds(0, TAIL)], gsem0).wait()
    pltpu.sync_copy(rows0_v.at[pl.ds(0, TAIL)], shacc.at[didxt_v], add=True)
    plsc.subcore_barrier()

    obase = pl.multiple_of(sid * 624, 8)

    @pl.when(sid < 15)
    def _cp():
        pltpu.sync_copy(shacc.at[pl.ds(obase, 624)],
                        out_hbm.at[cid, pl.ds(obase, 624)])

    @pl.when(sid == 15)
    def _cp_tail():
        pltpu.sync_copy(shacc.at[pl.ds(15 * 624, 640)],
                        out_hbm.at[cid, pl.ds(15 * 624, 640)])


@functools.cache
def _agg():
    return functools.partial(
        pl.kernel,
        mesh=plsc.VectorSubcoreMesh(**_sc_mesh),
        out_type=jax.ShapeDtypeStruct((2, N, E), jnp.float32),
        scratch_types=[
            pltpu.VMEM((EPT,), jnp.int32),
            pltpu.VMEM((CH,), jnp.int32),
            pltpu.VMEM((CH,), jnp.int32),
            pltpu.VMEM((TAIL,), jnp.int32),
            pltpu.VMEM((CH, E), jnp.float32),
            pltpu.VMEM((CH, E), jnp.float32),
            pltpu.VMEM_SHARED((NPAD, E), jnp.float32),
            pltpu.SemaphoreType.DMA,
            pltpu.SemaphoreType.DMA,
            pltpu.SemaphoreType.DMA,
            pltpu.SemaphoreType.DMA,
        ],
    )(_agg_body)


def _dense_body(aggp_ref, y_ref, dinvb_ref, batch_ref, seed_ref, qp_ref,
                wk_ref, bk_ref, wv_ref, bv_ref, ow_ref, ob_ref,
                fw_ref, fb_ref, l1g_ref, l1b_ref, l2g_ref, l2b_ref,
                out_ref, qt_ref, m_ref, den_ref, num_ref, cnt_ref):
    i = pl.program_id(0)
    f32 = jnp.float32

    @pl.when(i == 0)
    def _init():
        m_ref[...] = jnp.full((B, E), -1e30, f32)
        den_ref[...] = jnp.zeros((B, E), f32)
        num_ref[...] = jnp.zeros((B, E, E), f32)
        cnt_ref[...] = jnp.zeros((B, E), f32)
        q2 = lax.dot_general(seed_ref[...], qp_ref[...],
                             (((1,), (1,)), ((), ())),
                             preferred_element_type=f32) * (D_H ** -0.5)
        rep = jnp.concatenate([q2] * H, axis=0)
        rh = lax.broadcasted_iota(jnp.int32, (E, E), 0) // D_H
        eh = lax.broadcasted_iota(jnp.int32, (E, E), 1) // D_H
        qt_ref[...] = jnp.where(rh == eh, rep, 0.0)

    pre = dinvb_ref[...] * (aggp_ref[0] + aggp_ref[1] + y_ref[...])
    k = lax.dot_general(pre, wk_ref[...], (((1,), (1,)), ((), ())),
                        preferred_element_type=f32) + bk_ref[...]
    v = lax.dot_general(pre, wv_ref[...], (((1,), (1,)), ((), ())),
                        preferred_element_type=f32) + bv_ref[...]
    logits = lax.dot_general(k, qt_ref[...], (((1,), (1,)), ((), ())),
                             preferred_element_type=f32)

    b_blk = batch_ref[0, 0, :]
    gid = lax.broadcasted_iota(jnp.int32, (RB, B), 1)
    bm = b_blk[:, None] == gid
    bmf = bm.astype(f32)
    cnt_ref[...] = cnt_ref[...] + jnp.sum(bmf, axis=0)[:, None]

    for g in range(B):
        mk = bm[:, g:g + 1]

        @pl.when(jnp.sum(bmf[:, g]) > 0.0)
        def _upd(g=g, mk=mk):
            lg = jnp.where(mk, logits, -1e30)
            bmax = jnp.max(lg, axis=0)
            mold = m_ref[g, :]
            mnew = jnp.maximum(mold, bmax)
            adj = jnp.exp(mold - mnew)
            w = jnp.where(mk, jnp.exp(logits - mnew[None, :]), 0.0)
            den_ref[g, :] = den_ref[g, :] * adj + jnp.sum(w, axis=0)
            num_ref[g, :, :] = (num_ref[g, :, :] * adj[:, None] +
                                lax.dot_general(w, v,
                                                (((0,), (0,)), ((), ())),
                                                preferred_element_type=f32))
            m_ref[g, :] = mnew

    @pl.when(i == NBLK - 1)
    def _fin():
        counts = cnt_ref[...][:, 0]
        nmax = jnp.max(counts)
        pad = nmax - counts
        mfin = m_ref[...]
        padterm = jnp.where(pad[:, None] > 0.0,
                            pad[:, None] * jnp.exp(-mfin), 0.0)
        denf = den_ref[...] + padterm

        rows = []
        for g in range(B):
            segs = []
            for h in range(H):
                sub = num_ref[g, h * D_H:(h + 1) * D_H,
                              h * D_H:(h + 1) * D_H]
                dsub = denf[g, h * D_H:(h + 1) * D_H]
                segs.append(sub / dsub[:, None])
            rows.append(jnp.concatenate(segs, axis=1))
        ao = jnp.concatenate(rows, axis=0)

        qm = jnp.broadcast_to(seed_ref[...][None, :, :],
                              (B, L, E)).reshape(B * L, E)
        ao2 = lax.dot_general(ao, ow_ref[...], (((1,), (1,)), ((), ())),
                              preferred_element_type=f32) + ob_ref[...]
        s1 = qm + ao2
        mu1 = jnp.mean(s1, axis=1, keepdims=True)
        va1 = jnp.mean((s1 - mu1) ** 2, axis=1, keepdims=True)
        e1 = ((s1 - mu1) / jnp.sqrt(va1 + 1e-5)) * l1g_ref[...] + l1b_ref[...]
        ff = lax.dot_general(e1, fw_ref[...], (((1,), (1,)), ((), ())),
                             preferred_element_type=f32) + fb_ref[...]
        s2 = e1 + ff
        mu2 = jnp.mean(s2, axis=1, keepdims=True)
        va2 = jnp.mean((s2 - mu2) ** 2, axis=1, keepdims=True)
        e2 = ((s2 - mu2) / jnp.sqrt(va2 + 1e-5)) * l2g_ref[...] + l2b_ref[...]
        out_ref[...] = e2.reshape(B, L, E)


def _full(shape):
    nd = len(shape)
    return pl.BlockSpec(shape, lambda i, _n=nd: (0,) * _n)


def _dense(aggp, y, dinvb, batch3, seedf, qp, wkf, bkf, wvf, bvf,
           ow, ob, fw, fb, l1g, l1b, l2g, l2b):
    return pl.pallas_call(
        _dense_body,
        grid=(NBLK,),
        in_specs=[
            pl.BlockSpec((2, RB, E), lambda i: (0, i, 0)),
            pl.BlockSpec((RB, E), lambda i: (i, 0)),
            pl.BlockSpec((RB, E), lambda i: (i, 0)),
            pl.BlockSpec((1, 1, RB), lambda i: (i, 0, 0)),
            _full((L, E)), _full((E, E)),
            _full((E, E)), _full((1, E)), _full((E, E)), _full((1, E)),
            _full((E, E)), _full((1, E)), _full((E, E)), _full((1, E)),
            _full((1, E)), _full((1, E)), _full((1, E)), _full((1, E)),
        ],
        out_specs=pl.BlockSpec((B, L, E), lambda i: (0, 0, 0)),
        out_shape=jax.ShapeDtypeStruct((B, L, E), jnp.float32),
        scratch_shapes=[
            pltpu.VMEM((E, E), jnp.float32),
            pltpu.VMEM((B, E), jnp.float32),
            pltpu.VMEM((B, E), jnp.float32),
            pltpu.VMEM((B, E, E), jnp.float32),
            pltpu.VMEM((B, E), jnp.float32),
        ],
        compiler_params=pltpu.CompilerParams(
            dimension_semantics=("arbitrary",)),
    )(aggp, y, dinvb, batch3, seedf, qp, wkf, bkf, wvf, bvf,
      ow, ob, fw, fb, l1g, l1b, l2g, l2b)


def kernel(x_dense, x, edge_index, batch, seed, Wk, bk, Wv, bv,
           q_proj_weight, out_w, out_b, ff_w, ff_b,
           ln1_g, ln1_b, ln2_g, ln2_b):
    src = edge_index[0]
    dst = edge_index[1]

    deg_part = _deg()(dst)
    y, dinvb = _scale(deg_part, x)
    aggp = _agg()(src, dst, y)

    return _dense(
        aggp, y, dinvb,
        batch.reshape(NBLK, 1, RB),
        seed.reshape(L, E), q_proj_weight,
        Wk.reshape(E, E), bk.reshape(1, E),
        Wv.reshape(E, E), bv.reshape(1, E),
        out_w, out_b.reshape(1, E), ff_w, ff_b.reshape(1, E),
        ln1_g.reshape(1, E), ln1_b.reshape(1, E),
        ln2_g.reshape(1, E), ln2_b.reshape(1, E),
    )
